# baseline (speedup 1.0000x reference)
import functools

import jax
import jax.numpy as jnp
from jax import lax
from jax.experimental import pallas as pl
from jax.experimental.pallas import tpu as pltpu
from jax.experimental.pallas import tpu_sc as plsc


def kernel(img, index_flat_inv):
    B, T, C = img.shape
    idx = index_flat_inv.astype(jnp.int32)

    info = plsc.get_sparse_core_info()
    NC, NS = info.num_cores, info.num_subcores
    NW = NC * NS
    assert B == NW

    CH = 32
    NBUF = 4
    n_ch = T // CH
    idx2 = idx.reshape(n_ch, CH)

    mesh = plsc.VectorSubcoreMesh(core_axis_name="c", subcore_axis_name="s")

    @functools.partial(
        pl.kernel,
        mesh=mesh,
        out_type=jax.ShapeDtypeStruct((B, T, C), jnp.float32),
        scratch_types=(
            [pltpu.VMEM((n_ch, CH), jnp.int32)]
            + [pltpu.VMEM((CH, C), jnp.float32)] * NBUF
            + [pltpu.SemaphoreType.DMA] * (2 * NBUF)
        ),
    )
    def k(img_hbm, idx_hbm, out_hbm, perm_v, *rest):
        bufs = rest[:NBUF]
        rsems = rest[NBUF:2 * NBUF]
        wsems = rest[2 * NBUF:]
        wid = lax.axis_index("s") * NC + lax.axis_index("c")
        rd = [None] * NBUF
        wr = [None] * NBUF
        pltpu.sync_copy(idx_hbm, perm_v)
        for j in range(NBUF - 1):
            rd[j] = pltpu.async_copy(
                img_hbm.at[wid].at[perm_v.at[j]], bufs[j], rsems[j])
        for j in range(n_ch):
            cur = j % NBUF
            nj = j + NBUF - 1
            if nj < n_ch:
                b = nj % NBUF
                if wr[b] is not None:
                    wr[b].wait()
                    wr[b] = None
                rd[b] = pltpu.async_copy(
                    img_hbm.at[wid].at[perm_v.at[nj]], bufs[b], rsems[b])
            rd[cur].wait()
            wr[cur] = pltpu.async_copy(
                bufs[cur], out_hbm.at[wid, pl.ds(j * CH, CH)], wsems[cur])
        for w in wr:
            if w is not None:
                w.wait()

    return k(img, idx2)


# speedup vs baseline: 1.0102x; 1.0102x over previous
import functools

import jax
import jax.numpy as jnp
from jax import lax
from jax.experimental import pallas as pl
from jax.experimental.pallas import tpu as pltpu
from jax.experimental.pallas import tpu_sc as plsc


def kernel(img, index_flat_inv):
    B, T, C = img.shape
    idx = index_flat_inv.astype(jnp.int32)

    info = plsc.get_sparse_core_info()
    NC, NS = info.num_cores, info.num_subcores
    NW = NC * NS
    assert B == NW

    CH = 32
    NBUF = 4
    n_ch = T // CH
    idx2 = idx.reshape(n_ch, CH)

    mesh = plsc.VectorSubcoreMesh(core_axis_name="c", subcore_axis_name="s")

    @functools.partial(
        pl.kernel,
        mesh=mesh,
        out_type=jax.ShapeDtypeStruct((B, T, C), jnp.float32),
        scratch_types=(
            [pltpu.VMEM((n_ch, CH), jnp.int32)]
            + [pltpu.VMEM((CH, C), jnp.float32)] * NBUF
            + [pltpu.SemaphoreType.DMA] * (2 * NBUF)
        ),
    )
    def k(img_hbm, idx_hbm, out_hbm, perm_v, *rest):
        bufs = rest[:NBUF]
        rsems = rest[NBUF:2 * NBUF]
        wsems = rest[2 * NBUF:]
        wid = lax.axis_index("s") * NC + lax.axis_index("c")
        rd = [None] * NBUF
        wr = [None] * NBUF
        for j in range(NBUF - 1):
            rd[j] = pltpu.async_copy(
                img_hbm.at[wid, pl.ds(j * CH, CH)], bufs[j], rsems[j])
        pltpu.sync_copy(idx_hbm, perm_v)
        for j in range(n_ch):
            cur = j % NBUF
            nj = j + NBUF - 1
            if nj < n_ch:
                b = nj % NBUF
                if wr[b] is not None:
                    wr[b].wait()
                    wr[b] = None
                rd[b] = pltpu.async_copy(
                    img_hbm.at[wid, pl.ds(nj * CH, CH)], bufs[b], rsems[b])
            rd[cur].wait()
            wr[cur] = pltpu.async_copy(
                bufs[cur], out_hbm.at[wid].at[perm_v.at[j]], wsems[cur])
        for w in wr:
            if w is not None:
                w.wait()

    return k(img, idx2)
